# baseline (device time: 95146 ns/iter reference)
import jax
import jax.numpy as jnp
from jax import lax
from jax.experimental import pallas as pl
from jax.experimental.pallas import tpu as pltpu

N_DEV = 4
SQ = 1024
QBLK = SQ // N_DEV
WIN = 512
HQ = 8
DH = 128
DM = HQ * DH
WINDOW = 128
SCALE = 0.08838834764831843


def kernel(x, Wq, K_ext, V_ext, Wo):
    def body(x_ref, wq_ref, k_ref, v_ref, wo_ref, out_ref,
             k_stage, v_stage, k_win, v_win,
             kv_send_sems, kv_recv_sems, ag_send_sems, ag_recv_sems):
        my_i = lax.axis_index("i")

        for h in range(HQ):
            k_stage[:, DH * h:DH * (h + 1)] = k_ref[0, :, h, :].astype(jnp.bfloat16)
            v_stage[:, DH * h:DH * (h + 1)] = v_ref[0, :, h, :].astype(jnp.bfloat16)

        def rc(src, dst, ssem, rsem, dev):
            return pltpu.make_async_remote_copy(
                src_ref=src, dst_ref=dst, send_sem=ssem, recv_sem=rsem,
                device_id=(dev,), device_id_type=pl.DeviceIdType.MESH)

        flows01 = [
            rc(k_stage.at[pl.ds(128, 512)], k_win.at[pl.ds(0, 512)],
               kv_send_sems.at[0], kv_recv_sems.at[0], 1),
            rc(v_stage.at[pl.ds(128, 512)], v_win.at[pl.ds(0, 512)],
               kv_send_sems.at[1], kv_recv_sems.at[1], 1),
        ]
        flows02 = [
            rc(k_stage.at[pl.ds(384, 512)], k_win.at[pl.ds(0, 512)],
               kv_send_sems.at[2], kv_recv_sems.at[0], 2),
            rc(v_stage.at[pl.ds(384, 512)], v_win.at[pl.ds(0, 512)],
               kv_send_sems.at[3], kv_recv_sems.at[1], 2),
        ]
        flows03 = [
            rc(k_stage.at[pl.ds(640, 384)], k_win.at[pl.ds(0, 384)],
               kv_send_sems.at[4], kv_recv_sems.at[0], 3),
            rc(v_stage.at[pl.ds(640, 384)], v_win.at[pl.ds(0, 384)],
               kv_send_sems.at[5], kv_recv_sems.at[1], 3),
        ]
        flows13 = [
            rc(k_stage.at[pl.ds(0, 128)], k_win.at[pl.ds(384, 128)],
               kv_send_sems.at[6], kv_recv_sems.at[2], 3),
            rc(v_stage.at[pl.ds(0, 128)], v_win.at[pl.ds(384, 128)],
               kv_send_sems.at[7], kv_recv_sems.at[3], 3),
        ]

        @pl.when(my_i == 0)
        def _():
            for f in flows01 + flows02 + flows03:
                f.start()
            k_win[...] = k_stage[0:WIN, :]
            v_win[...] = v_stage[0:WIN, :]

        @pl.when(my_i == 1)
        def _():
            for f in flows13:
                f.start()

        xb = x_ref[0, pl.ds(QBLK * my_i, QBLK), :].astype(jnp.bfloat16)
        wqb = wq_ref[...].astype(jnp.bfloat16)
        qb = lax.dot_general(xb, wqb, (((1,), (0,)), ((), ())),
                             preferred_element_type=jnp.float32
                             ).astype(jnp.bfloat16)

        @pl.when((my_i == 1) | (my_i == 2))
        def _():
            flows01[0].wait_recv()
            flows01[1].wait_recv()

        @pl.when(my_i == 3)
        def _():
            for f in flows03 + flows13:
                f.wait_recv()

        wstart = jnp.maximum(QBLK * my_i - WINDOW, 0)
        qpos = QBLK * my_i + lax.broadcasted_iota(jnp.int32, (QBLK, WIN), 0)
        kpos = wstart + lax.broadcasted_iota(jnp.int32, (QBLK, WIN), 1)
        mask = jnp.abs(qpos - kpos) <= WINDOW

        kw = k_win[...]
        vw = v_win[...]
        ctx_parts = []
        for h in range(HQ):
            qh = qb[:, DH * h:DH * (h + 1)]
            kh = kw[:, DH * h:DH * (h + 1)]
            vh = vw[:, DH * h:DH * (h + 1)]
            s = lax.dot_general(qh, kh, (((1,), (1,)), ((), ())),
                                preferred_element_type=jnp.float32) * SCALE
            s = jnp.where(mask, s, -1e9)
            m = jnp.max(s, axis=1, keepdims=True)
            w = jnp.exp(s - m)
            w = w / jnp.sum(w, axis=1, keepdims=True)
            ctx_parts.append(
                lax.dot_general(w.astype(jnp.bfloat16), vh,
                                (((1,), (0,)), ((), ())),
                                preferred_element_type=jnp.float32))
        ctx = jnp.concatenate(ctx_parts, axis=1).astype(jnp.bfloat16)
        wob = wo_ref[...].astype(jnp.bfloat16)
        orow = lax.dot_general(ctx, wob, (((1,), (0,)), ((), ())),
                               preferred_element_type=jnp.float32)
        out_ref[0, pl.ds(QBLK * my_i, QBLK), :] = orow

        ag = []
        for k in range(1, N_DEV):
            dst = lax.rem(my_i + k, N_DEV)
            ag.append(rc(out_ref.at[0, pl.ds(QBLK * my_i, QBLK), :],
                         out_ref.at[0, pl.ds(QBLK * my_i, QBLK), :],
                         ag_send_sems.at[k], ag_recv_sems.at[k], dst))
        for a in ag:
            a.start()
        for k in range(1, N_DEV):
            srcdev = lax.rem(my_i - k + N_DEV, N_DEV)
            rcv = rc(out_ref.at[0, pl.ds(QBLK * srcdev, QBLK), :],
                     out_ref.at[0, pl.ds(QBLK * srcdev, QBLK), :],
                     ag_send_sems.at[k], ag_recv_sems.at[k], srcdev)
            rcv.wait_recv()
        for a in ag:
            a.wait_send()

        @pl.when(my_i == 0)
        def _():
            for f in flows01 + flows02 + flows03:
                f.wait_send()

        @pl.when(my_i == 1)
        def _():
            for f in flows13:
                f.wait_send()

    return pl.pallas_call(
        body,
        out_shape=jax.ShapeDtypeStruct((1, SQ, DM), jnp.float32),
        in_specs=[pl.BlockSpec(memory_space=pltpu.VMEM)] * 5,
        out_specs=pl.BlockSpec(memory_space=pltpu.VMEM),
        scratch_shapes=[
            pltpu.VMEM((1024, DM), jnp.bfloat16),
            pltpu.VMEM((1024, DM), jnp.bfloat16),
            pltpu.VMEM((WIN, DM), jnp.bfloat16),
            pltpu.VMEM((WIN, DM), jnp.bfloat16),
            pltpu.SemaphoreType.DMA((8,)),
            pltpu.SemaphoreType.DMA((4,)),
            pltpu.SemaphoreType.DMA((4,)),
            pltpu.SemaphoreType.DMA((4,)),
        ],
    )(x, Wq, K_ext, V_ext, Wo)


# device time: 69917 ns/iter; 1.3608x vs baseline; 1.3608x over previous
import jax
import jax.numpy as jnp
from jax import lax
from jax.experimental import pallas as pl
from jax.experimental.pallas import tpu as pltpu

N_DEV = 4
SQ = 1024
QBLK = SQ // N_DEV
WIN = 512
HQ = 8
DH = 128
DM = HQ * DH
WINDOW = 128
SCALE = 0.08838834764831843


def kernel(x, Wq, K_ext, V_ext, Wo):
    def body(x_ref, wq_ref, k_ref, v_ref, wo_ref, out_ref,
             k_stage, v_stage, k_win, v_win, ag_buf,
             snd0, snd, rcv, ag_snd, ag_rcv):
        my_i = lax.axis_index("i")

        def stage_chunk(lo, n):
            for h in range(HQ):
                k_stage[lo:lo + n, DH * h:DH * (h + 1)] = (
                    k_ref[0, lo:lo + n, h, :].astype(jnp.bfloat16))
                v_stage[lo:lo + n, DH * h:DH * (h + 1)] = (
                    v_ref[0, lo:lo + n, h, :].astype(jnp.bfloat16))

        def rc(src, dst, ssem, rsem, dev):
            return pltpu.make_async_remote_copy(
                src_ref=src, dst_ref=dst, send_sem=ssem, recv_sem=rsem,
                device_id=(dev,), device_id_type=pl.DeviceIdType.MESH)

        def kv_pair(slo, n, dlo, ssems, si, rsems, ri, dev,
                    src_k=None, src_v=None):
            sk = (src_k if src_k is not None else k_stage).at[pl.ds(slo, n)]
            sv = (src_v if src_v is not None else v_stage).at[pl.ds(slo, n)]
            return (rc(sk, k_win.at[pl.ds(dlo, n)], ssems.at[si], rsems.at[ri], dev),
                    rc(sv, v_win.at[pl.ds(dlo, n)], ssems.at[si + 1], rsems.at[ri + 1], dev))

        f1a = kv_pair(384, 256, 256, snd0, 0, rcv, 0, 1)
        f1b = kv_pair(128, 256, 0, snd0, 2, rcv, 2, 1)
        f2a = kv_pair(640, 256, 0, snd0, 4, rcv, 0, 3)
        f2b = kv_pair(896, 128, 256, snd0, 6, rcv, 2, 3)
        f3 = kv_pair(0, 128, 384, snd, 0, rcv, 4, 3)
        f4 = kv_pair(256, 256, 0, snd, 2, rcv, 0, 2, src_k=k_win, src_v=v_win)
        f5 = kv_pair(0, 256, 256, snd, 0, rcv, 2, 2, src_k=k_win, src_v=v_win)

        @pl.when(my_i == 0)
        def _():
            stage_chunk(384, 256)
            f1a[0].start(); f1a[1].start()
            stage_chunk(640, 256)
            f2a[0].start(); f2a[1].start()
            stage_chunk(128, 256)
            f1b[0].start(); f1b[1].start()
            stage_chunk(896, 128)
            f2b[0].start(); f2b[1].start()
            stage_chunk(0, 128)
            k_win[...] = k_stage[0:WIN, :]
            v_win[...] = v_stage[0:WIN, :]

        @pl.when(my_i == 1)
        def _():
            stage_chunk(0, 128)
            f3[0].start(); f3[1].start()

        xb = x_ref[0, pl.ds(QBLK * my_i, QBLK), :].astype(jnp.bfloat16)
        wqb = wq_ref[...].astype(jnp.bfloat16)
        qb = lax.dot_general(xb, wqb, (((1,), (0,)), ((), ())),
                             preferred_element_type=jnp.float32
                             ).astype(jnp.bfloat16)

        @pl.when(my_i == 1)
        def _():
            f1a[0].wait_recv(); f1a[1].wait_recv()
            f4[0].start(); f4[1].start()
            f1b[0].wait_recv(); f1b[1].wait_recv()

        @pl.when(my_i == 3)
        def _():
            f2a[0].wait_recv(); f2a[1].wait_recv()
            f5[0].start(); f5[1].start()
            f2b[0].wait_recv(); f2b[1].wait_recv()
            f3[0].wait_recv(); f3[1].wait_recv()

        @pl.when(my_i == 2)
        def _():
            for f in (f4, f5):
                f[0].wait_recv(); f[1].wait_recv()

        wstart = jnp.maximum(QBLK * my_i - WINDOW, 0)
        qpos = QBLK * my_i + lax.broadcasted_iota(jnp.int32, (QBLK, WIN), 0)
        kpos = wstart + lax.broadcasted_iota(jnp.int32, (QBLK, WIN), 1)
        mask = jnp.abs(qpos - kpos) <= WINDOW

        kw = k_win[...]
        vw = v_win[...]
        ctx_parts = []
        for h in range(HQ):
            qh = qb[:, DH * h:DH * (h + 1)]
            kh = kw[:, DH * h:DH * (h + 1)]
            vh = vw[:, DH * h:DH * (h + 1)]
            s = lax.dot_general(qh, kh, (((1,), (1,)), ((), ())),
                                preferred_element_type=jnp.float32) * SCALE
            s = jnp.where(mask, s, -1e9)
            m = jnp.max(s, axis=1, keepdims=True)
            w = jnp.exp(s - m)
            w = w / jnp.sum(w, axis=1, keepdims=True)
            ctx_parts.append(
                lax.dot_general(w.astype(jnp.bfloat16), vh,
                                (((1,), (0,)), ((), ())),
                                preferred_element_type=jnp.float32))
        ctx = jnp.concatenate(ctx_parts, axis=1).astype(jnp.bfloat16)
        wob = wo_ref[...].astype(jnp.bfloat16)
        orow = lax.dot_general(ctx, wob, (((1,), (0,)), ((), ())),
                               preferred_element_type=jnp.float32)

        ag_buf[pl.ds(QBLK * my_i, QBLK), :] = orow.astype(jnp.bfloat16)
        ag = []
        for k in range(1, N_DEV):
            dst = lax.rem(my_i + k, N_DEV)
            ag.append(rc(ag_buf.at[pl.ds(QBLK * my_i, QBLK), :],
                         ag_buf.at[pl.ds(QBLK * my_i, QBLK), :],
                         ag_snd.at[k], ag_rcv.at[k], dst))
        for a in ag:
            a.start()
        for k in range(1, N_DEV):
            srcdev = lax.rem(my_i - k + N_DEV, N_DEV)
            rcvd = rc(ag_buf.at[pl.ds(QBLK * srcdev, QBLK), :],
                      ag_buf.at[pl.ds(QBLK * srcdev, QBLK), :],
                      ag_snd.at[k], ag_rcv.at[k], srcdev)
            rcvd.wait_recv()
        out_ref[0, :, :] = ag_buf[...].astype(jnp.float32)
        for a in ag:
            a.wait_send()

        @pl.when(my_i == 0)
        def _():
            for f in (f1a, f2a, f1b, f2b):
                f[0].wait_send(); f[1].wait_send()

        @pl.when(my_i == 1)
        def _():
            for f in (f3, f4):
                f[0].wait_send(); f[1].wait_send()

        @pl.when(my_i == 3)
        def _():
            f5[0].wait_send(); f5[1].wait_send()

    return pl.pallas_call(
        body,
        out_shape=jax.ShapeDtypeStruct((1, SQ, DM), jnp.float32),
        in_specs=[pl.BlockSpec(memory_space=pltpu.VMEM)] * 5,
        out_specs=pl.BlockSpec(memory_space=pltpu.VMEM),
        scratch_shapes=[
            pltpu.VMEM((1024, DM), jnp.bfloat16),
            pltpu.VMEM((1024, DM), jnp.bfloat16),
            pltpu.VMEM((WIN, DM), jnp.bfloat16),
            pltpu.VMEM((WIN, DM), jnp.bfloat16),
            pltpu.VMEM((SQ, DM), jnp.bfloat16),
            pltpu.SemaphoreType.DMA((8,)),
            pltpu.SemaphoreType.DMA((4,)),
            pltpu.SemaphoreType.DMA((6,)),
            pltpu.SemaphoreType.DMA((4,)),
            pltpu.SemaphoreType.DMA((4,)),
        ],
    )(x, Wq, K_ext, V_ext, Wo)
